# baseline (speedup 1.0000x reference)
import jax
import jax.numpy as jnp
from jax import lax
from jax.experimental import pallas as pl
from jax.experimental.pallas import tpu as pltpu
from jax.experimental.pallas import tpu_sc as plsc

N_CHANNELS = 384
N_NEIGHB = 64
N_SPIKES = 1_000_000
N_QUERY_CH = 96
MIN_COVERAGE = 0.9

L = 16
NC = 2
NS = 16
NW = NC * NS
NV = N_SPIKES // L
BASE_V = NV // NW
REM_V = NV % NW
CHUNK_V = 512
CW = CHUNK_V * L
N_CHUNKS = -(-(BASE_V + 1) // CHUNK_V)
U = 16


def _sc_body(ind_hbm, ids_hbm, ch_hbm, pc_hbm,
             cov_hbm, cvd_hbm, nsp_hbm, out_hbm,
             ind_v, ch_v, pc_v, small_v, table_v,
             id0, id1, ob0, ob1, shared_tbl,
             sin0, sin1, sout0, sout1):
    cid = lax.axis_index("c")
    sid = lax.axis_index("s")
    iota = lax.iota(jnp.int32, L)

    w = sid * NC + cid
    n_w = BASE_V + jnp.where(w < REM_V, 1, 0)
    s_w = BASE_V * w + jnp.minimum(w, REM_V)

    idbufs = [id0, id1]
    obufs = [ob0, ob1]
    sins = [sin0, sin1]
    souts = [sout0, sout1]

    def chunk_base(i):
        coff = jnp.minimum(jnp.int32(i * CHUNK_V), n_w - CHUNK_V)
        return (s_w + coff) * L

    in_d = [None] * N_CHUNKS
    in_d[0] = pltpu.async_copy(ids_hbm.at[pl.ds(chunk_base(0), CW)], id0, sin0)
    in_d[1] = pltpu.async_copy(ids_hbm.at[pl.ds(chunk_base(1), CW)], id1, sin1)

    @pl.when(sid == 0)
    def _phase_a():
        pltpu.sync_copy(ind_hbm, ind_v)
        pltpu.sync_copy(ch_hbm, ch_v)
        pltpu.sync_copy(pc_hbm, pc_v)
        zero = jnp.zeros((L,), jnp.float32)

        def cnt_body(c, accs):
            accs = list(accs)
            for u in range(4):
                for jj in range(4):
                    accs[jj] = accs[jj] + ind_v[c * 4 + u, pl.ds(jj * L, L)]
            return tuple(accs)
        cnts = lax.fori_loop(0, N_CHANNELS // 4, cnt_body, (zero,) * 4)

        def row_body(k, accs):
            g = k // L
            r = k % L
            chr_ = plsc.load_gather(ch_v, [g * L + ((iota + r) & (L - 1))])
            accs = list(accs)
            for jj in range(4):
                accs[jj] = accs[jj] + plsc.load_gather(
                    ind_v, [chr_, jj * L + iota])
            return tuple(accs)
        ssums = lax.fori_loop(0, N_QUERY_CH, row_body, (zero,) * 4)

        nsp = jnp.int32(0)
        for jj in range(4):
            cov = ssums[jj] / cnts[jj]
            cvd = cov >= MIN_COVERAGE
            masked = jnp.where(cvd, cov, jnp.float32(0.0))
            pc = pc_v[pl.ds(jj * L, L)]
            nsp = nsp + jnp.sum(jnp.where(cvd, pc, jnp.int32(0)))
            small_v[pl.ds(jj * L, L)] = cov
            small_v[pl.ds(N_NEIGHB + jj * L, L)] = jnp.where(
                cvd, jnp.float32(1.0), jnp.float32(0.0))
            table_v[pl.ds(jj * L, L)] = masked
        small_v[pl.ds(2 * N_NEIGHB, L)] = jnp.full(
            (L,), nsp, jnp.int32).astype(jnp.float32)
        pltpu.sync_copy(table_v, shared_tbl)

        @pl.when(cid == 0)
        def _write_small():
            pltpu.sync_copy(small_v.at[pl.ds(0, N_NEIGHB)], cov_hbm)
            pltpu.sync_copy(small_v.at[pl.ds(N_NEIGHB, N_NEIGHB)], cvd_hbm)
            pltpu.sync_copy(small_v.at[pl.ds(2 * N_NEIGHB, L)], nsp_hbm)

    with jax.named_scope("barrier"):
        plsc.subcore_barrier()
        pltpu.sync_copy(shared_tbl, table_v)

    out_d = [None] * N_CHUNKS
    for i in range(N_CHUNKS):
        ib = idbufs[i % 2]
        ob = obufs[i % 2]
        with jax.named_scope(f"wait{i}"):
            if i >= 2:
                out_d[i - 2].wait()
            in_d[i].wait()

        def g_body(k, carry, ib=ib, ob=ob):
            for u in range(U):
                off = (k * U + u) * L
                ob[pl.ds(off, L)] = plsc.load_gather(table_v, [ib[pl.ds(off, L)]])
            return carry
        with jax.named_scope(f"gather{i}"):
            lax.fori_loop(0, CHUNK_V // U, g_body, jnp.int32(0))

        if i + 2 < N_CHUNKS:
            in_d[i + 2] = pltpu.async_copy(
                ids_hbm.at[pl.ds(chunk_base(i + 2), CW)], ib, sins[i % 2])
        out_d[i] = pltpu.async_copy(
            ob, out_hbm.at[pl.ds(chunk_base(i), CW)], souts[i % 2])
    with jax.named_scope("drain"):
        out_d[N_CHUNKS - 2].wait()
        out_d[N_CHUNKS - 1].wait()


@jax.jit
def _run(ind, ids, ch, pc):
    mesh = plsc.VectorSubcoreMesh(core_axis_name="c", subcore_axis_name="s",
                                  num_cores=NC, num_subcores=NS)
    f = pl.kernel(
        _sc_body,
        out_type=(
            jax.ShapeDtypeStruct((N_NEIGHB,), jnp.float32),
            jax.ShapeDtypeStruct((N_NEIGHB,), jnp.float32),
            jax.ShapeDtypeStruct((L,), jnp.float32),
            jax.ShapeDtypeStruct((N_SPIKES,), jnp.float32),
        ),
        mesh=mesh,
        compiler_params=pltpu.CompilerParams(needs_layout_passes=False),
        scratch_types=(
            pltpu.VMEM((N_CHANNELS, N_NEIGHB), jnp.float32),
            pltpu.VMEM((N_QUERY_CH,), jnp.int32),
            pltpu.VMEM((N_NEIGHB,), jnp.int32),
            pltpu.VMEM((2 * N_NEIGHB + L,), jnp.float32),
            pltpu.VMEM((N_NEIGHB,), jnp.float32),
            pltpu.VMEM((CW,), jnp.int32),
            pltpu.VMEM((CW,), jnp.int32),
            pltpu.VMEM((CW,), jnp.float32),
            pltpu.VMEM((CW,), jnp.float32),
            pltpu.VMEM_SHARED((N_NEIGHB,), jnp.float32),
            pltpu.SemaphoreType.DMA,
            pltpu.SemaphoreType.DMA,
            pltpu.SemaphoreType.DMA,
            pltpu.SemaphoreType.DMA,
        ),
    )
    return f(ind, ids, ch, pc)


def kernel(indicators, neighborhood_ids, channels, popcounts):
    cov, cvd, nsp, spike_cov = _run(
        indicators.astype(jnp.float32), neighborhood_ids.astype(jnp.int32),
        channels.astype(jnp.int32), popcounts.astype(jnp.int32))
    covered = cvd != 0.0
    n_spikes_covered = nsp[0].astype(jnp.int32)
    return cov, covered, n_spikes_covered, spike_cov


# speedup vs baseline: 1.0754x; 1.0754x over previous
import jax
import jax.numpy as jnp
from jax import lax
from jax.experimental import pallas as pl
from jax.experimental.pallas import tpu as pltpu
from jax.experimental.pallas import tpu_sc as plsc

N_CHANNELS = 384
N_NEIGHB = 64
N_SPIKES = 1_000_000
N_QUERY_CH = 96
MIN_COVERAGE = 0.9

L = 16
NC = 2
NS = 16
NW = NC * NS
NV = N_SPIKES // L
BASE_V = NV // NW
REM_V = NV % NW
CHUNK_V = 512
CW = CHUNK_V * L
N_CHUNKS = -(-(BASE_V + 1) // CHUNK_V)
U = 16
RPT = N_CHANNELS // NS


def _sc_body(ind_hbm, ids_hbm, ch_hbm, pc_hbm,
             cov_hbm, cvd_hbm, nsp_hbm, out_hbm,
             ind24_v, ch_v, pc_v, m32_v, part_v, acc_v, idx8_v, zero_v,
             small_v, table_v, id0, id1, ob0, ob1, shared_acc,
             sin0, sin1, sout0, sout1):
    cid = lax.axis_index("c")
    sid = lax.axis_index("s")
    iota = lax.iota(jnp.int32, L)
    zero = jnp.zeros((L,), jnp.float32)

    w = sid * NC + cid
    n_w = BASE_V + jnp.where(w < REM_V, 1, 0)
    s_w = BASE_V * w + jnp.minimum(w, REM_V)

    idbufs = [id0, id1]
    obufs = [ob0, ob1]
    sins = [sin0, sin1]
    souts = [sout0, sout1]

    def chunk_base(i):
        coff = jnp.minimum(jnp.int32(i * CHUNK_V), n_w - CHUNK_V)
        return (s_w + coff) * L

    in_d = [None] * N_CHUNKS
    in_d[0] = pltpu.async_copy(ids_hbm.at[pl.ds(chunk_base(0), CW)], id0, sin0)
    in_d[1] = pltpu.async_copy(ids_hbm.at[pl.ds(chunk_base(1), CW)], id1, sin1)

    plsc.store_scatter(idx8_v, [iota], iota, mask=iota < 8)
    pltpu.sync_copy(ind_hbm.at[pl.ds(sid * RPT, RPT)], ind24_v)
    pltpu.sync_copy(ch_hbm, ch_v)

    @pl.when(sid == 0)
    def _zero_shared():
        for i in range(8):
            zero_v[i, pl.ds(0, L)] = zero
        pltpu.sync_copy(zero_v, shared_acc)

    @pl.when((sid == 0) & (cid == 0))
    def _load_pc():
        pltpu.sync_copy(pc_hbm, pc_v)

    m32_v[pl.ds(0, L)] = zero
    m32_v[pl.ds(L, L)] = zero
    ones = jnp.ones((L,), jnp.float32)
    base_row = sid * RPT
    for g in range(N_QUERY_CH // L):
        local = ch_v[pl.ds(g * L, L)] - base_row
        msk = (local >= 0) & (local < RPT)
        plsc.addupdate_scatter(
            m32_v, [jnp.clip(local, 0, 2 * L - 1)], ones, mask=msk)

    cnt = [zero] * 4
    ss = [zero] * 4
    for r in range(RPT):
        mb = plsc.load_gather(m32_v, [jnp.full((L,), r, jnp.int32)])
        for jj in range(4):
            row = ind24_v[r, pl.ds(jj * L, L)]
            cnt[jj] = cnt[jj] + row
            ss[jj] = ss[jj] + mb * row
    for jj in range(4):
        part_v[jj, pl.ds(0, L)] = cnt[jj]
        part_v[4 + jj, pl.ds(0, L)] = ss[jj]

    with jax.named_scope("combine"):
        plsc.subcore_barrier()
        pltpu.sync_copy(part_v, shared_acc.at[idx8_v], add=True)
        plsc.subcore_barrier()
        pltpu.sync_copy(shared_acc, acc_v)

    for jj in range(4):
        covj = acc_v[4 + jj, pl.ds(0, L)] / acc_v[jj, pl.ds(0, L)]
        cvdj = covj >= MIN_COVERAGE
        table_v[pl.ds(jj * L, L)] = jnp.where(cvdj, covj, jnp.float32(0.0))

    @pl.when((sid == 0) & (cid == 0))
    def _write_small():
        nsp = jnp.int32(0)
        for jj in range(4):
            covj = acc_v[4 + jj, pl.ds(0, L)] / acc_v[jj, pl.ds(0, L)]
            cvdj = covj >= MIN_COVERAGE
            pc = pc_v[pl.ds(jj * L, L)]
            nsp = nsp + jnp.sum(jnp.where(cvdj, pc, jnp.int32(0)))
            small_v[pl.ds(jj * L, L)] = covj
            small_v[pl.ds(N_NEIGHB + jj * L, L)] = jnp.where(
                cvdj, jnp.float32(1.0), jnp.float32(0.0))
        small_v[pl.ds(2 * N_NEIGHB, L)] = jnp.full(
            (L,), nsp, jnp.int32).astype(jnp.float32)
        pltpu.sync_copy(small_v.at[pl.ds(0, N_NEIGHB)], cov_hbm)
        pltpu.sync_copy(small_v.at[pl.ds(N_NEIGHB, N_NEIGHB)], cvd_hbm)
        pltpu.sync_copy(small_v.at[pl.ds(2 * N_NEIGHB, L)], nsp_hbm)

    out_d = [None] * N_CHUNKS
    for i in range(N_CHUNKS):
        ib = idbufs[i % 2]
        ob = obufs[i % 2]
        with jax.named_scope(f"wait{i}"):
            if i >= 2:
                out_d[i - 2].wait()
            in_d[i].wait()

        def g_body(k, carry, ib=ib, ob=ob):
            for u in range(U):
                off = (k * U + u) * L
                ob[pl.ds(off, L)] = plsc.load_gather(table_v, [ib[pl.ds(off, L)]])
            return carry
        with jax.named_scope(f"gather{i}"):
            lax.fori_loop(0, CHUNK_V // U, g_body, jnp.int32(0))

        if i + 2 < N_CHUNKS:
            in_d[i + 2] = pltpu.async_copy(
                ids_hbm.at[pl.ds(chunk_base(i + 2), CW)], ib, sins[i % 2])
        out_d[i] = pltpu.async_copy(
            ob, out_hbm.at[pl.ds(chunk_base(i), CW)], souts[i % 2])
    with jax.named_scope("drain"):
        out_d[N_CHUNKS - 2].wait()
        out_d[N_CHUNKS - 1].wait()


@jax.jit
def _run(ind, ids, ch, pc):
    mesh = plsc.VectorSubcoreMesh(core_axis_name="c", subcore_axis_name="s",
                                  num_cores=NC, num_subcores=NS)
    f = pl.kernel(
        _sc_body,
        out_type=(
            jax.ShapeDtypeStruct((N_NEIGHB,), jnp.float32),
            jax.ShapeDtypeStruct((N_NEIGHB,), jnp.float32),
            jax.ShapeDtypeStruct((L,), jnp.float32),
            jax.ShapeDtypeStruct((N_SPIKES,), jnp.float32),
        ),
        mesh=mesh,
        compiler_params=pltpu.CompilerParams(needs_layout_passes=False),
        scratch_types=(
            pltpu.VMEM((RPT, N_NEIGHB), jnp.float32),
            pltpu.VMEM((N_QUERY_CH,), jnp.int32),
            pltpu.VMEM((N_NEIGHB,), jnp.int32),
            pltpu.VMEM((2 * L,), jnp.float32),
            pltpu.VMEM((8, L), jnp.float32),
            pltpu.VMEM((8, L), jnp.float32),
            pltpu.VMEM((8,), jnp.int32),
            pltpu.VMEM((8, L), jnp.float32),
            pltpu.VMEM((2 * N_NEIGHB + L,), jnp.float32),
            pltpu.VMEM((N_NEIGHB,), jnp.float32),
            pltpu.VMEM((CW,), jnp.int32),
            pltpu.VMEM((CW,), jnp.int32),
            pltpu.VMEM((CW,), jnp.float32),
            pltpu.VMEM((CW,), jnp.float32),
            pltpu.VMEM_SHARED((8, L), jnp.float32),
            pltpu.SemaphoreType.DMA,
            pltpu.SemaphoreType.DMA,
            pltpu.SemaphoreType.DMA,
            pltpu.SemaphoreType.DMA,
        ),
    )
    return f(ind, ids, ch, pc)


def kernel(indicators, neighborhood_ids, channels, popcounts):
    cov, cvd, nsp, spike_cov = _run(
        indicators.astype(jnp.float32), neighborhood_ids.astype(jnp.int32),
        channels.astype(jnp.int32), popcounts.astype(jnp.int32))
    covered = cvd != 0.0
    n_spikes_covered = nsp[0].astype(jnp.int32)
    return cov, covered, n_spikes_covered, spike_cov


# speedup vs baseline: 1.2640x; 1.1754x over previous
import jax
import jax.numpy as jnp
from jax import lax
from jax.experimental import pallas as pl
from jax.experimental.pallas import tpu as pltpu
from jax.experimental.pallas import tpu_sc as plsc

N_CHANNELS = 384
N_NEIGHB = 64
N_SPIKES = 1_000_000
N_QUERY_CH = 96
MIN_COVERAGE = 0.9

L = 16
NC = 2
NS = 16
NW = NC * NS
NV = N_SPIKES // L
BASE_V = NV // NW
REM_V = NV % NW
CHUNK_V = 512
CW = CHUNK_V * L
N_CHUNKS = -(-(BASE_V + 1) // CHUNK_V)
U = 16
RPT = N_CHANNELS // NS


def _sc_body(ind_hbm, ids_hbm, ch_hbm, pc_hbm,
             cov_hbm, cvd_hbm, nsp_hbm, out_hbm,
             ind24_v, ch_v, pc_v, m32_v, part_v, acc_v, idx8_v, zero_v,
             small_v, table_v, id0, id1, ob0, ob1, shared_acc,
             sin0, sin1, sout0, sout1):
    cid = lax.axis_index("c")
    sid = lax.axis_index("s")
    iota = lax.iota(jnp.int32, L)
    zero = jnp.zeros((L,), jnp.float32)

    w = sid * NC + cid
    n_w = BASE_V + jnp.where(w < REM_V, 1, 0)
    s_w = BASE_V * w + jnp.minimum(w, REM_V)

    idbufs = [id0, id1]
    obufs = [ob0, ob1]
    sins = [sin0, sin1]
    souts = [sout0, sout1]

    def chunk_base(i):
        coff = jnp.minimum(jnp.int32(i * CHUNK_V), n_w - CHUNK_V)
        return (s_w + coff) * L

    in_d = [None] * N_CHUNKS
    in_d[0] = pltpu.async_copy(ids_hbm.at[pl.ds(chunk_base(0), CW)], id0, sin0)
    in_d[1] = pltpu.async_copy(ids_hbm.at[pl.ds(chunk_base(1), CW)], id1, sin1)

    plsc.store_scatter(idx8_v, [iota], iota, mask=iota < 8)
    pltpu.sync_copy(ind_hbm.at[pl.ds(sid * RPT, RPT)], ind24_v)
    pltpu.sync_copy(ch_hbm, ch_v)

    @pl.when(sid == 0)
    def _zero_shared():
        for i in range(8):
            zero_v[i, pl.ds(0, L)] = zero
        pltpu.sync_copy(zero_v, shared_acc)

    @pl.when((sid == 0) & (cid == 0))
    def _load_pc():
        pltpu.sync_copy(pc_hbm, pc_v)

    m32_v[pl.ds(0, L)] = zero
    m32_v[pl.ds(L, L)] = zero
    ones = jnp.ones((L,), jnp.float32)
    base_row = sid * RPT
    for g in range(N_QUERY_CH // L):
        local = ch_v[pl.ds(g * L, L)] - base_row
        msk = (local >= 0) & (local < RPT)
        plsc.addupdate_scatter(
            m32_v, [jnp.clip(local, 0, 2 * L - 1)], ones, mask=msk)

    cnt = [zero] * 4
    ss = [zero] * 4
    for r in range(RPT):
        mb = plsc.load_gather(m32_v, [jnp.full((L,), r, jnp.int32)])
        for jj in range(4):
            row = ind24_v[r, pl.ds(jj * L, L)]
            cnt[jj] = cnt[jj] + row
            ss[jj] = ss[jj] + mb * row
    for jj in range(4):
        part_v[jj, pl.ds(0, L)] = cnt[jj]
        part_v[4 + jj, pl.ds(0, L)] = ss[jj]

    with jax.named_scope("combine"):
        plsc.subcore_barrier()
        pltpu.sync_copy(part_v, shared_acc.at[idx8_v], add=True)
        plsc.subcore_barrier()
        pltpu.sync_copy(shared_acc, acc_v)

    for jj in range(4):
        covj = acc_v[4 + jj, pl.ds(0, L)] / acc_v[jj, pl.ds(0, L)]
        cvdj = covj >= MIN_COVERAGE
        table_v[pl.ds(jj * L, L)] = jnp.where(cvdj, covj, jnp.float32(0.0))

    @pl.when((sid == 0) & (cid == 0))
    def _write_small():
        nsp = jnp.int32(0)
        for jj in range(4):
            covj = acc_v[4 + jj, pl.ds(0, L)] / acc_v[jj, pl.ds(0, L)]
            cvdj = covj >= MIN_COVERAGE
            pc = pc_v[pl.ds(jj * L, L)]
            nsp = nsp + jnp.sum(jnp.where(cvdj, pc, jnp.int32(0)))
            small_v[pl.ds(jj * L, L)] = covj
            small_v[pl.ds(N_NEIGHB + jj * L, L)] = jnp.where(
                cvdj, jnp.float32(1.0), jnp.float32(0.0))
        small_v[pl.ds(2 * N_NEIGHB, L)] = jnp.full(
            (L,), nsp, jnp.int32).astype(jnp.float32)
        pltpu.sync_copy(small_v.at[pl.ds(0, N_NEIGHB)], cov_hbm)
        pltpu.sync_copy(small_v.at[pl.ds(N_NEIGHB, N_NEIGHB)], cvd_hbm)
        pltpu.sync_copy(small_v.at[pl.ds(2 * N_NEIGHB, L)], nsp_hbm)

    out_d = [None] * N_CHUNKS
    for i in range(N_CHUNKS):
        ib = idbufs[i % 2]
        ob = obufs[i % 2]
        with jax.named_scope(f"wait{i}"):
            if i >= 2:
                out_d[i - 2].wait()
            in_d[i].wait()

        with jax.named_scope(f"gather{i}"):
            @plsc.parallel_loop(0, CHUNK_V, step=1, unroll=U)
            def _g(k, ib=ib, ob=ob):
                off = k * L
                ob[pl.ds(off, L)] = plsc.load_gather(
                    table_v, [ib[pl.ds(off, L)]])

        if i + 2 < N_CHUNKS:
            in_d[i + 2] = pltpu.async_copy(
                ids_hbm.at[pl.ds(chunk_base(i + 2), CW)], ib, sins[i % 2])
        out_d[i] = pltpu.async_copy(
            ob, out_hbm.at[pl.ds(chunk_base(i), CW)], souts[i % 2])
    with jax.named_scope("drain"):
        out_d[N_CHUNKS - 2].wait()
        out_d[N_CHUNKS - 1].wait()


@jax.jit
def _run(ind, ids, ch, pc):
    mesh = plsc.VectorSubcoreMesh(core_axis_name="c", subcore_axis_name="s",
                                  num_cores=NC, num_subcores=NS)
    f = pl.kernel(
        _sc_body,
        out_type=(
            jax.ShapeDtypeStruct((N_NEIGHB,), jnp.float32),
            jax.ShapeDtypeStruct((N_NEIGHB,), jnp.float32),
            jax.ShapeDtypeStruct((L,), jnp.float32),
            jax.ShapeDtypeStruct((N_SPIKES,), jnp.float32),
        ),
        mesh=mesh,
        compiler_params=pltpu.CompilerParams(needs_layout_passes=False),
        scratch_types=(
            pltpu.VMEM((RPT, N_NEIGHB), jnp.float32),
            pltpu.VMEM((N_QUERY_CH,), jnp.int32),
            pltpu.VMEM((N_NEIGHB,), jnp.int32),
            pltpu.VMEM((2 * L,), jnp.float32),
            pltpu.VMEM((8, L), jnp.float32),
            pltpu.VMEM((8, L), jnp.float32),
            pltpu.VMEM((8,), jnp.int32),
            pltpu.VMEM((8, L), jnp.float32),
            pltpu.VMEM((2 * N_NEIGHB + L,), jnp.float32),
            pltpu.VMEM((N_NEIGHB,), jnp.float32),
            pltpu.VMEM((CW,), jnp.int32),
            pltpu.VMEM((CW,), jnp.int32),
            pltpu.VMEM((CW,), jnp.float32),
            pltpu.VMEM((CW,), jnp.float32),
            pltpu.VMEM_SHARED((8, L), jnp.float32),
            pltpu.SemaphoreType.DMA,
            pltpu.SemaphoreType.DMA,
            pltpu.SemaphoreType.DMA,
            pltpu.SemaphoreType.DMA,
        ),
    )
    return f(ind, ids, ch, pc)


def kernel(indicators, neighborhood_ids, channels, popcounts):
    cov, cvd, nsp, spike_cov = _run(
        indicators.astype(jnp.float32), neighborhood_ids.astype(jnp.int32),
        channels.astype(jnp.int32), popcounts.astype(jnp.int32))
    covered = cvd != 0.0
    n_spikes_covered = nsp[0].astype(jnp.int32)
    return cov, covered, n_spikes_covered, spike_cov


# speedup vs baseline: 1.2861x; 1.0175x over previous
import jax
import jax.numpy as jnp
from jax import lax
from jax.experimental import pallas as pl
from jax.experimental.pallas import tpu as pltpu
from jax.experimental.pallas import tpu_sc as plsc

N_CHANNELS = 384
N_NEIGHB = 64
N_SPIKES = 1_000_000
N_QUERY_CH = 96
MIN_COVERAGE = 0.9

L = 16
NC = 2
NS = 16
NW = NC * NS
NV = N_SPIKES // L
BASE_V = NV // NW
REM_V = NV % NW
CHUNK_V = 512
CW = CHUNK_V * L
N_CHUNKS = -(-(BASE_V + 1) // CHUNK_V)
U = 16
RPT = N_CHANNELS // NS


def _sc_body(ind_hbm, ids_hbm, ch_hbm, pc_hbm,
             cov_hbm, cvd_hbm, nsp_hbm, out_hbm,
             ind24_v, ch_v, pc_v, m32_v, part_v, acc_v, idx8_v, zero_v,
             small_v, table_v, idbufs, obufs, shared_acc,
             sins, souts, sa):
    cid = lax.axis_index("c")
    sid = lax.axis_index("s")
    iota = lax.iota(jnp.int32, L)
    zero = jnp.zeros((L,), jnp.float32)

    w = sid * NC + cid
    n_w = BASE_V + jnp.where(w < REM_V, 1, 0)
    s_w = BASE_V * w + jnp.minimum(w, REM_V)

    def chunk_base(i):
        coff = jnp.minimum(jnp.int32(i * CHUNK_V), n_w - CHUNK_V)
        return (s_w + coff) * L

    in_d = [pltpu.async_copy(ids_hbm.at[pl.ds(chunk_base(i), CW)],
                             idbufs[i], sins[i])
            for i in range(N_CHUNKS)]

    ind_d = pltpu.async_copy(ind_hbm.at[pl.ds(sid * RPT, RPT)], ind24_v, sa)
    ch_d = pltpu.async_copy(ch_hbm, ch_v, sa)
    plsc.store_scatter(idx8_v, [iota], iota, mask=iota < 8)

    @pl.when(sid == 0)
    def _zero_shared():
        for i in range(8):
            zero_v[i, pl.ds(0, L)] = zero
        pltpu.sync_copy(zero_v, shared_acc)

    @pl.when((sid == 0) & (cid == 0))
    def _load_pc():
        pltpu.sync_copy(pc_hbm, pc_v)

    m32_v[pl.ds(0, L)] = zero
    m32_v[pl.ds(L, L)] = zero
    ones = jnp.ones((L,), jnp.float32)
    base_row = sid * RPT
    ind_d.wait()
    ch_d.wait()
    for g in range(N_QUERY_CH // L):
        local = ch_v[pl.ds(g * L, L)] - base_row
        msk = (local >= 0) & (local < RPT)
        plsc.addupdate_scatter(
            m32_v, [jnp.clip(local, 0, 2 * L - 1)], ones, mask=msk)

    @plsc.parallel_loop(0, RPT, step=1, unroll=6, carry=(zero,) * 8)
    def sums(r, accs):
        mb = plsc.load_gather(m32_v, [jnp.full((L,), r, jnp.int32)])
        accs = list(accs)
        for jj in range(4):
            row = ind24_v[r, pl.ds(jj * L, L)]
            accs[jj] = accs[jj] + row
            accs[4 + jj] = accs[4 + jj] + mb * row
        return tuple(accs)
    for jj in range(4):
        part_v[jj, pl.ds(0, L)] = sums[jj]
        part_v[4 + jj, pl.ds(0, L)] = sums[4 + jj]

    with jax.named_scope("combine"):
        plsc.subcore_barrier()
        pltpu.sync_copy(part_v, shared_acc.at[idx8_v], add=True)
        plsc.subcore_barrier()
        pltpu.sync_copy(shared_acc, acc_v)

    for jj in range(4):
        covj = acc_v[4 + jj, pl.ds(0, L)] / acc_v[jj, pl.ds(0, L)]
        cvdj = covj >= MIN_COVERAGE
        table_v[pl.ds(jj * L, L)] = jnp.where(cvdj, covj, jnp.float32(0.0))

    @pl.when((sid == 0) & (cid == 0))
    def _write_small():
        nsp = jnp.int32(0)
        for jj in range(4):
            covj = acc_v[4 + jj, pl.ds(0, L)] / acc_v[jj, pl.ds(0, L)]
            cvdj = covj >= MIN_COVERAGE
            pc = pc_v[pl.ds(jj * L, L)]
            nsp = nsp + jnp.sum(jnp.where(cvdj, pc, jnp.int32(0)))
            small_v[pl.ds(jj * L, L)] = covj
            small_v[pl.ds(N_NEIGHB + jj * L, L)] = jnp.where(
                cvdj, jnp.float32(1.0), jnp.float32(0.0))
        small_v[pl.ds(2 * N_NEIGHB, L)] = jnp.full(
            (L,), nsp, jnp.int32).astype(jnp.float32)
        pltpu.sync_copy(small_v.at[pl.ds(0, N_NEIGHB)], cov_hbm)
        pltpu.sync_copy(small_v.at[pl.ds(N_NEIGHB, N_NEIGHB)], cvd_hbm)
        pltpu.sync_copy(small_v.at[pl.ds(2 * N_NEIGHB, L)], nsp_hbm)

    out_d = [None] * N_CHUNKS
    for i in range(N_CHUNKS):
        ib = idbufs[i]
        ob = obufs[i]
        with jax.named_scope(f"wait{i}"):
            in_d[i].wait()

        with jax.named_scope(f"gather{i}"):
            @plsc.parallel_loop(0, CHUNK_V, step=1, unroll=U)
            def _g(k, ib=ib, ob=ob):
                off = k * L
                ob[pl.ds(off, L)] = plsc.load_gather(
                    table_v, [ib[pl.ds(off, L)]])

        out_d[i] = pltpu.async_copy(
            ob, out_hbm.at[pl.ds(chunk_base(i), CW)], souts[i])
    with jax.named_scope("drain"):
        for i in range(N_CHUNKS):
            out_d[i].wait()


@jax.jit
def _run(ind, ids, ch, pc):
    mesh = plsc.VectorSubcoreMesh(core_axis_name="c", subcore_axis_name="s",
                                  num_cores=NC, num_subcores=NS)
    f = pl.kernel(
        _sc_body,
        out_type=(
            jax.ShapeDtypeStruct((N_NEIGHB,), jnp.float32),
            jax.ShapeDtypeStruct((N_NEIGHB,), jnp.float32),
            jax.ShapeDtypeStruct((L,), jnp.float32),
            jax.ShapeDtypeStruct((N_SPIKES,), jnp.float32),
        ),
        mesh=mesh,
        compiler_params=pltpu.CompilerParams(needs_layout_passes=False),
        scratch_types=(
            pltpu.VMEM((RPT, N_NEIGHB), jnp.float32),
            pltpu.VMEM((N_QUERY_CH,), jnp.int32),
            pltpu.VMEM((N_NEIGHB,), jnp.int32),
            pltpu.VMEM((2 * L,), jnp.float32),
            pltpu.VMEM((8, L), jnp.float32),
            pltpu.VMEM((8, L), jnp.float32),
            pltpu.VMEM((8,), jnp.int32),
            pltpu.VMEM((8, L), jnp.float32),
            pltpu.VMEM((2 * N_NEIGHB + L,), jnp.float32),
            pltpu.VMEM((N_NEIGHB,), jnp.float32),
            [pltpu.VMEM((CW,), jnp.int32)] * N_CHUNKS,
            [pltpu.VMEM((CW,), jnp.float32)] * N_CHUNKS,
            pltpu.VMEM_SHARED((8, L), jnp.float32),
            [pltpu.SemaphoreType.DMA] * N_CHUNKS,
            [pltpu.SemaphoreType.DMA] * N_CHUNKS,
            pltpu.SemaphoreType.DMA,
        ),
    )
    return f(ind, ids, ch, pc)


def kernel(indicators, neighborhood_ids, channels, popcounts):
    cov, cvd, nsp, spike_cov = _run(
        indicators.astype(jnp.float32), neighborhood_ids.astype(jnp.int32),
        channels.astype(jnp.int32), popcounts.astype(jnp.int32))
    covered = cvd != 0.0
    n_spikes_covered = nsp[0].astype(jnp.int32)
    return cov, covered, n_spikes_covered, spike_cov


# speedup vs baseline: 1.2905x; 1.0034x over previous
import jax
import jax.numpy as jnp
from jax import lax
from jax.experimental import pallas as pl
from jax.experimental.pallas import tpu as pltpu
from jax.experimental.pallas import tpu_sc as plsc

N_CHANNELS = 384
N_NEIGHB = 64
N_SPIKES = 1_000_000
N_QUERY_CH = 96
MIN_COVERAGE = 0.9

L = 16
NC = 2
NS = 16
NW = NC * NS
NV = N_SPIKES // L
BASE_V = NV // NW
REM_V = NV % NW
CHUNK_V = 512
CW = CHUNK_V * L
N_CHUNKS = -(-(BASE_V + 1) // CHUNK_V)
U = 16
RPT = N_CHANNELS // NS


def _sc_body(ind_hbm, ids_hbm, ch_hbm, pc_hbm,
             cov_hbm, cvd_hbm, nsp_hbm, out_hbm,
             ind24_v, ch_v, pc_v, m32_v, part_v, acc_v, idx8_v, zero_v,
             small_v, table_v, idbufs, obufs, shared_acc,
             sins, souts, sa):
    cid = lax.axis_index("c")
    sid = lax.axis_index("s")
    iota = lax.iota(jnp.int32, L)
    zero = jnp.zeros((L,), jnp.float32)

    w = sid * NC + cid
    n_w = BASE_V + jnp.where(w < REM_V, 1, 0)
    s_w = BASE_V * w + jnp.minimum(w, REM_V)

    def chunk_base(i):
        coff = jnp.minimum(jnp.int32(i * CHUNK_V), n_w - CHUNK_V)
        return (s_w + coff) * L

    ind_d = pltpu.async_copy(ind_hbm.at[pl.ds(sid * RPT, RPT)], ind24_v, sa)
    ch_d = pltpu.async_copy(ch_hbm, ch_v, sa)
    in_d = [pltpu.async_copy(ids_hbm.at[pl.ds(chunk_base(i), CW)],
                             idbufs[i], sins[i])
            for i in range(N_CHUNKS)]
    plsc.store_scatter(idx8_v, [iota], iota, mask=iota < 8)

    @pl.when(sid == 0)
    def _zero_shared():
        for i in range(8):
            zero_v[i, pl.ds(0, L)] = zero
        pltpu.sync_copy(zero_v, shared_acc)

    @pl.when((sid == 0) & (cid == 0))
    def _load_pc():
        pltpu.sync_copy(pc_hbm, pc_v)

    m32_v[pl.ds(0, L)] = zero
    m32_v[pl.ds(L, L)] = zero
    ones = jnp.ones((L,), jnp.float32)
    base_row = sid * RPT
    ind_d.wait()
    ch_d.wait()
    for g in range(N_QUERY_CH // L):
        local = ch_v[pl.ds(g * L, L)] - base_row
        msk = (local >= 0) & (local < RPT)
        plsc.addupdate_scatter(
            m32_v, [jnp.clip(local, 0, 2 * L - 1)], ones, mask=msk)

    @plsc.parallel_loop(0, RPT, step=1, unroll=6, carry=(zero,) * 8)
    def sums(r, accs):
        mb = plsc.load_gather(m32_v, [jnp.full((L,), r, jnp.int32)])
        accs = list(accs)
        for jj in range(4):
            row = ind24_v[r, pl.ds(jj * L, L)]
            accs[jj] = accs[jj] + row
            accs[4 + jj] = accs[4 + jj] + mb * row
        return tuple(accs)
    for jj in range(4):
        part_v[jj, pl.ds(0, L)] = sums[jj]
        part_v[4 + jj, pl.ds(0, L)] = sums[4 + jj]

    with jax.named_scope("combine"):
        plsc.subcore_barrier()
        pltpu.sync_copy(part_v, shared_acc.at[idx8_v], add=True)
        plsc.subcore_barrier()
        pltpu.sync_copy(shared_acc, acc_v)

    for jj in range(4):
        covj = acc_v[4 + jj, pl.ds(0, L)] / acc_v[jj, pl.ds(0, L)]
        cvdj = covj >= MIN_COVERAGE
        table_v[pl.ds(jj * L, L)] = jnp.where(cvdj, covj, jnp.float32(0.0))

    out_d = [None] * N_CHUNKS
    for i in range(N_CHUNKS):
        ib = idbufs[i]
        ob = obufs[i]
        with jax.named_scope(f"wait{i}"):
            in_d[i].wait()

        with jax.named_scope(f"gather{i}"):
            @plsc.parallel_loop(0, CHUNK_V, step=1, unroll=U)
            def _g(k, ib=ib, ob=ob):
                off = k * L
                ob[pl.ds(off, L)] = plsc.load_gather(
                    table_v, [ib[pl.ds(off, L)]])

        out_d[i] = pltpu.async_copy(
            ob, out_hbm.at[pl.ds(chunk_base(i), CW)], souts[i])

    @pl.when((sid == 0) & (cid == 0))
    def _write_small():
        nsp = jnp.int32(0)
        for jj in range(4):
            covj = acc_v[4 + jj, pl.ds(0, L)] / acc_v[jj, pl.ds(0, L)]
            cvdj = covj >= MIN_COVERAGE
            pc = pc_v[pl.ds(jj * L, L)]
            nsp = nsp + jnp.sum(jnp.where(cvdj, pc, jnp.int32(0)))
            small_v[pl.ds(jj * L, L)] = covj
            small_v[pl.ds(N_NEIGHB + jj * L, L)] = jnp.where(
                cvdj, jnp.float32(1.0), jnp.float32(0.0))
        small_v[pl.ds(2 * N_NEIGHB, L)] = jnp.full(
            (L,), nsp, jnp.int32).astype(jnp.float32)
        pltpu.sync_copy(small_v.at[pl.ds(0, N_NEIGHB)], cov_hbm)
        pltpu.sync_copy(small_v.at[pl.ds(N_NEIGHB, N_NEIGHB)], cvd_hbm)
        pltpu.sync_copy(small_v.at[pl.ds(2 * N_NEIGHB, L)], nsp_hbm)

    with jax.named_scope("drain"):
        for i in range(N_CHUNKS):
            out_d[i].wait()


@jax.jit
def _run(ind, ids, ch, pc):
    mesh = plsc.VectorSubcoreMesh(core_axis_name="c", subcore_axis_name="s",
                                  num_cores=NC, num_subcores=NS)
    f = pl.kernel(
        _sc_body,
        out_type=(
            jax.ShapeDtypeStruct((N_NEIGHB,), jnp.float32),
            jax.ShapeDtypeStruct((N_NEIGHB,), jnp.float32),
            jax.ShapeDtypeStruct((L,), jnp.float32),
            jax.ShapeDtypeStruct((N_SPIKES,), jnp.float32),
        ),
        mesh=mesh,
        compiler_params=pltpu.CompilerParams(needs_layout_passes=False),
        scratch_types=(
            pltpu.VMEM((RPT, N_NEIGHB), jnp.float32),
            pltpu.VMEM((N_QUERY_CH,), jnp.int32),
            pltpu.VMEM((N_NEIGHB,), jnp.int32),
            pltpu.VMEM((2 * L,), jnp.float32),
            pltpu.VMEM((8, L), jnp.float32),
            pltpu.VMEM((8, L), jnp.float32),
            pltpu.VMEM((8,), jnp.int32),
            pltpu.VMEM((8, L), jnp.float32),
            pltpu.VMEM((2 * N_NEIGHB + L,), jnp.float32),
            pltpu.VMEM((N_NEIGHB,), jnp.float32),
            [pltpu.VMEM((CW,), jnp.int32)] * N_CHUNKS,
            [pltpu.VMEM((CW,), jnp.float32)] * N_CHUNKS,
            pltpu.VMEM_SHARED((8, L), jnp.float32),
            [pltpu.SemaphoreType.DMA] * N_CHUNKS,
            [pltpu.SemaphoreType.DMA] * N_CHUNKS,
            pltpu.SemaphoreType.DMA,
        ),
    )
    return f(ind, ids, ch, pc)


def kernel(indicators, neighborhood_ids, channels, popcounts):
    cov, cvd, nsp, spike_cov = _run(
        indicators.astype(jnp.float32), neighborhood_ids.astype(jnp.int32),
        channels.astype(jnp.int32), popcounts.astype(jnp.int32))
    covered = cvd != 0.0
    n_spikes_covered = nsp[0].astype(jnp.int32)
    return cov, covered, n_spikes_covered, spike_cov


# speedup vs baseline: 1.2906x; 1.0001x over previous
import jax
import jax.numpy as jnp
from jax import lax
from jax.experimental import pallas as pl
from jax.experimental.pallas import tpu as pltpu
from jax.experimental.pallas import tpu_sc as plsc

N_CHANNELS = 384
N_NEIGHB = 64
N_SPIKES = 1_000_000
N_QUERY_CH = 96
MIN_COVERAGE = 0.9

L = 16
NC = 2
NS = 16
NW = NC * NS
NV = N_SPIKES // L
BASE_V = NV // NW
REM_V = NV % NW
CHUNK_V = 512
CW = CHUNK_V * L
N_CHUNKS = -(-(BASE_V + 1) // CHUNK_V)
U = 16
RPT = N_CHANNELS // NS


def _sc_body(ind_hbm, ids_hbm, ch_hbm, pc_hbm,
             cov_hbm, cvd_hbm, nsp_hbm, out_hbm,
             ind24_v, ch_v, pc_v, m32_v, part_v, acc_v, idx8_v, zero_v,
             small_v, table_v, idbufs, obufs, shared_acc,
             sins, souts, sa):
    cid = lax.axis_index("c")
    sid = lax.axis_index("s")
    iota = lax.iota(jnp.int32, L)
    zero = jnp.zeros((L,), jnp.float32)

    w = sid * NC + cid
    n_w = BASE_V + jnp.where(w < REM_V, 1, 0)
    s_w = BASE_V * w + jnp.minimum(w, REM_V)

    def chunk_base(i):
        coff = jnp.minimum(jnp.int32(i * CHUNK_V), n_w - CHUNK_V)
        return (s_w + coff) * L

    ind_d = pltpu.async_copy(ind_hbm.at[pl.ds(sid * RPT, RPT)], ind24_v, sa)
    ch_d = pltpu.async_copy(ch_hbm, ch_v, sa)
    in_d = [pltpu.async_copy(ids_hbm.at[pl.ds(chunk_base(i), CW)],
                             idbufs[i], sins[i])
            for i in range(N_CHUNKS)]
    plsc.store_scatter(idx8_v, [iota], iota, mask=iota < 8)

    @pl.when(sid == 0)
    def _zero_shared():
        for i in range(8):
            zero_v[i, pl.ds(0, L)] = zero
        pltpu.sync_copy(zero_v, shared_acc)

    @pl.when((sid == 0) & (cid == 0))
    def _load_pc():
        pltpu.sync_copy(pc_hbm, pc_v)

    m32_v[pl.ds(0, L)] = zero
    m32_v[pl.ds(L, L)] = zero
    ones = jnp.ones((L,), jnp.float32)
    base_row = sid * RPT
    with jax.named_scope("ainputs"):
        ind_d.wait()
        ch_d.wait()
    with jax.named_scope("mbuild"):
        for g in range(N_QUERY_CH // L):
            local = ch_v[pl.ds(g * L, L)] - base_row
            msk = (local >= 0) & (local < RPT)
            plsc.addupdate_scatter(
                m32_v, [jnp.clip(local, 0, 2 * L - 1)], ones, mask=msk)

    with jax.named_scope("rowsums"):
        @plsc.parallel_loop(0, RPT, step=1, unroll=6, carry=(zero,) * 8)
        def sums(r, accs):
            mb = plsc.load_gather(m32_v, [jnp.full((L,), r, jnp.int32)])
            accs = list(accs)
            for jj in range(4):
                row = ind24_v[r, pl.ds(jj * L, L)]
                accs[jj] = accs[jj] + row
                accs[4 + jj] = accs[4 + jj] + mb * row
            return tuple(accs)
    for jj in range(4):
        part_v[jj, pl.ds(0, L)] = sums[jj]
        part_v[4 + jj, pl.ds(0, L)] = sums[4 + jj]

    with jax.named_scope("combine"):
        plsc.subcore_barrier()
        pltpu.sync_copy(part_v, shared_acc.at[idx8_v], add=True)
        plsc.subcore_barrier()
        pltpu.sync_copy(shared_acc, acc_v)

    for jj in range(4):
        covj = acc_v[4 + jj, pl.ds(0, L)] / acc_v[jj, pl.ds(0, L)]
        cvdj = covj >= MIN_COVERAGE
        table_v[pl.ds(jj * L, L)] = jnp.where(cvdj, covj, jnp.float32(0.0))

    out_d = [None] * N_CHUNKS
    for i in range(N_CHUNKS):
        ib = idbufs[i]
        ob = obufs[i]
        with jax.named_scope(f"wait{i}"):
            in_d[i].wait()

        with jax.named_scope(f"gather{i}"):
            @plsc.parallel_loop(0, CHUNK_V, step=1, unroll=U)
            def _g(k, ib=ib, ob=ob):
                off = k * L
                ob[pl.ds(off, L)] = plsc.load_gather(
                    table_v, [ib[pl.ds(off, L)]])

        out_d[i] = pltpu.async_copy(
            ob, out_hbm.at[pl.ds(chunk_base(i), CW)], souts[i])

    @pl.when((sid == 0) & (cid == 0))
    def _write_small():
        nsp = jnp.int32(0)
        for jj in range(4):
            covj = acc_v[4 + jj, pl.ds(0, L)] / acc_v[jj, pl.ds(0, L)]
            cvdj = covj >= MIN_COVERAGE
            pc = pc_v[pl.ds(jj * L, L)]
            nsp = nsp + jnp.sum(jnp.where(cvdj, pc, jnp.int32(0)))
            small_v[pl.ds(jj * L, L)] = covj
            small_v[pl.ds(N_NEIGHB + jj * L, L)] = jnp.where(
                cvdj, jnp.float32(1.0), jnp.float32(0.0))
        small_v[pl.ds(2 * N_NEIGHB, L)] = jnp.full(
            (L,), nsp, jnp.int32).astype(jnp.float32)
        pltpu.sync_copy(small_v.at[pl.ds(0, N_NEIGHB)], cov_hbm)
        pltpu.sync_copy(small_v.at[pl.ds(N_NEIGHB, N_NEIGHB)], cvd_hbm)
        pltpu.sync_copy(small_v.at[pl.ds(2 * N_NEIGHB, L)], nsp_hbm)

    with jax.named_scope("drain"):
        for i in range(N_CHUNKS):
            out_d[i].wait()


@jax.jit
def _run(ind, ids, ch, pc):
    mesh = plsc.VectorSubcoreMesh(core_axis_name="c", subcore_axis_name="s",
                                  num_cores=NC, num_subcores=NS)
    f = pl.kernel(
        _sc_body,
        out_type=(
            jax.ShapeDtypeStruct((N_NEIGHB,), jnp.float32),
            jax.ShapeDtypeStruct((N_NEIGHB,), jnp.float32),
            jax.ShapeDtypeStruct((L,), jnp.float32),
            jax.ShapeDtypeStruct((N_SPIKES,), jnp.float32),
        ),
        mesh=mesh,
        compiler_params=pltpu.CompilerParams(needs_layout_passes=False),
        scratch_types=(
            pltpu.VMEM((RPT, N_NEIGHB), jnp.float32),
            pltpu.VMEM((N_QUERY_CH,), jnp.int32),
            pltpu.VMEM((N_NEIGHB,), jnp.int32),
            pltpu.VMEM((2 * L,), jnp.float32),
            pltpu.VMEM((8, L), jnp.float32),
            pltpu.VMEM((8, L), jnp.float32),
            pltpu.VMEM((8,), jnp.int32),
            pltpu.VMEM((8, L), jnp.float32),
            pltpu.VMEM((2 * N_NEIGHB + L,), jnp.float32),
            pltpu.VMEM((N_NEIGHB,), jnp.float32),
            [pltpu.VMEM((CW,), jnp.int32)] * N_CHUNKS,
            [pltpu.VMEM((CW,), jnp.float32)] * N_CHUNKS,
            pltpu.VMEM_SHARED((8, L), jnp.float32),
            [pltpu.SemaphoreType.DMA] * N_CHUNKS,
            [pltpu.SemaphoreType.DMA] * N_CHUNKS,
            pltpu.SemaphoreType.DMA,
        ),
    )
    return f(ind, ids, ch, pc)


def kernel(indicators, neighborhood_ids, channels, popcounts):
    cov, cvd, nsp, spike_cov = _run(
        indicators.astype(jnp.float32), neighborhood_ids.astype(jnp.int32),
        channels.astype(jnp.int32), popcounts.astype(jnp.int32))
    covered = cvd != 0.0
    n_spikes_covered = nsp[0].astype(jnp.int32)
    return cov, covered, n_spikes_covered, spike_cov


# speedup vs baseline: 1.3123x; 1.0168x over previous
import jax
import jax.numpy as jnp
from jax import lax
from jax.experimental import pallas as pl
from jax.experimental.pallas import tpu as pltpu
from jax.experimental.pallas import tpu_sc as plsc

N_CHANNELS = 384
N_NEIGHB = 64
N_SPIKES = 1_000_000
N_QUERY_CH = 96
MIN_COVERAGE = 0.9

L = 16
NC = 2
NS = 16
NW = NC * NS
NV = N_SPIKES // L
BASE_V = NV // NW
REM_V = NV % NW
CHUNK_V = 512
CW = CHUNK_V * L
N_CHUNKS = -(-(BASE_V + 1) // CHUNK_V)
TAIL_V = (BASE_V + 1) - (N_CHUNKS - 1) * CHUNK_V
CHUNK_SIZES = [CHUNK_V] * (N_CHUNKS - 1) + [TAIL_V]
U = 8
RPT = N_CHANNELS // NS


def _sc_body(ind_hbm, ids_hbm, ch_hbm, pc_hbm,
             cov_hbm, cvd_hbm, nsp_hbm, out_hbm,
             ind24_v, ch_v, pc_v, m32_v, part_v, acc_v, idx8_v, zero_v,
             small_v, table_v, idbufs, obufs, shared_acc,
             sins, souts, sa):
    cid = lax.axis_index("c")
    sid = lax.axis_index("s")
    iota = lax.iota(jnp.int32, L)
    zero = jnp.zeros((L,), jnp.float32)

    w = sid * NC + cid
    n_w = BASE_V + jnp.where(w < REM_V, 1, 0)
    s_w = BASE_V * w + jnp.minimum(w, REM_V)

    def chunk_base(i):
        if i < N_CHUNKS - 1:
            coff = jnp.int32(i * CHUNK_V)
        else:
            coff = n_w - TAIL_V
        return (s_w + coff) * L

    ind_d = pltpu.async_copy(ind_hbm.at[pl.ds(sid * RPT, RPT)], ind24_v, sa)
    ch_d = pltpu.async_copy(ch_hbm, ch_v, sa)
    in_d = [pltpu.async_copy(
        ids_hbm.at[pl.ds(chunk_base(i), CHUNK_SIZES[i] * L)],
        idbufs[i].at[pl.ds(0, CHUNK_SIZES[i] * L)], sins[i])
            for i in range(N_CHUNKS)]
    plsc.store_scatter(idx8_v, [iota], iota, mask=iota < 8)

    @pl.when(sid == 0)
    def _zero_shared():
        for i in range(8):
            zero_v[i, pl.ds(0, L)] = zero
        pltpu.sync_copy(zero_v, shared_acc)

    @pl.when((sid == 0) & (cid == 0))
    def _load_pc():
        pltpu.sync_copy(pc_hbm, pc_v)

    m32_v[pl.ds(0, L)] = zero
    m32_v[pl.ds(L, L)] = zero
    ones = jnp.ones((L,), jnp.float32)
    base_row = sid * RPT
    ind_d.wait()
    ch_d.wait()
    for g in range(N_QUERY_CH // L):
        local = ch_v[pl.ds(g * L, L)] - base_row
        msk = (local >= 0) & (local < RPT)
        plsc.addupdate_scatter(
            m32_v, [jnp.clip(local, 0, 2 * L - 1)], ones, mask=msk)

    @plsc.parallel_loop(0, RPT, step=1, unroll=6, carry=(zero,) * 8)
    def sums(r, accs):
        mb = plsc.load_gather(m32_v, [jnp.full((L,), r, jnp.int32)])
        accs = list(accs)
        for jj in range(4):
            row = ind24_v[r, pl.ds(jj * L, L)]
            accs[jj] = accs[jj] + row
            accs[4 + jj] = accs[4 + jj] + mb * row
        return tuple(accs)
    for jj in range(4):
        part_v[jj, pl.ds(0, L)] = sums[jj]
        part_v[4 + jj, pl.ds(0, L)] = sums[4 + jj]

    plsc.subcore_barrier()
    pltpu.sync_copy(part_v, shared_acc.at[idx8_v], add=True)
    plsc.subcore_barrier()
    pltpu.sync_copy(shared_acc, acc_v)

    for jj in range(4):
        covj = acc_v[4 + jj, pl.ds(0, L)] / acc_v[jj, pl.ds(0, L)]
        cvdj = covj >= MIN_COVERAGE
        table_v[pl.ds(jj * L, L)] = jnp.where(cvdj, covj, jnp.float32(0.0))

    out_d = [None] * N_CHUNKS
    for i in range(N_CHUNKS):
        ib = idbufs[i]
        ob = obufs[i]
        in_d[i].wait()

        @plsc.parallel_loop(0, CHUNK_SIZES[i], step=1, unroll=U)
        def _g(k, ib=ib, ob=ob):
            off = k * L
            ob[pl.ds(off, L)] = plsc.load_gather(
                table_v, [ib[pl.ds(off, L)]])

        out_d[i] = pltpu.async_copy(
            ob.at[pl.ds(0, CHUNK_SIZES[i] * L)],
            out_hbm.at[pl.ds(chunk_base(i), CHUNK_SIZES[i] * L)], souts[i])

    @pl.when((sid == 0) & (cid == 0))
    def _write_small():
        nsp = jnp.int32(0)
        for jj in range(4):
            covj = acc_v[4 + jj, pl.ds(0, L)] / acc_v[jj, pl.ds(0, L)]
            cvdj = covj >= MIN_COVERAGE
            pc = pc_v[pl.ds(jj * L, L)]
            nsp = nsp + jnp.sum(jnp.where(cvdj, pc, jnp.int32(0)))
            small_v[pl.ds(jj * L, L)] = covj
            small_v[pl.ds(N_NEIGHB + jj * L, L)] = jnp.where(
                cvdj, jnp.float32(1.0), jnp.float32(0.0))
        small_v[pl.ds(2 * N_NEIGHB, L)] = jnp.full(
            (L,), nsp, jnp.int32).astype(jnp.float32)
        pltpu.sync_copy(small_v.at[pl.ds(0, N_NEIGHB)], cov_hbm)
        pltpu.sync_copy(small_v.at[pl.ds(N_NEIGHB, N_NEIGHB)], cvd_hbm)
        pltpu.sync_copy(small_v.at[pl.ds(2 * N_NEIGHB, L)], nsp_hbm)

    for i in range(N_CHUNKS):
        out_d[i].wait()


@jax.jit
def _run(ind, ids, ch, pc):
    mesh = plsc.VectorSubcoreMesh(core_axis_name="c", subcore_axis_name="s",
                                  num_cores=NC, num_subcores=NS)
    f = pl.kernel(
        _sc_body,
        out_type=(
            jax.ShapeDtypeStruct((N_NEIGHB,), jnp.float32),
            jax.ShapeDtypeStruct((N_NEIGHB,), jnp.float32),
            jax.ShapeDtypeStruct((L,), jnp.float32),
            jax.ShapeDtypeStruct((N_SPIKES,), jnp.float32),
        ),
        mesh=mesh,
        compiler_params=pltpu.CompilerParams(needs_layout_passes=False),
        scratch_types=(
            pltpu.VMEM((RPT, N_NEIGHB), jnp.float32),
            pltpu.VMEM((N_QUERY_CH,), jnp.int32),
            pltpu.VMEM((N_NEIGHB,), jnp.int32),
            pltpu.VMEM((2 * L,), jnp.float32),
            pltpu.VMEM((8, L), jnp.float32),
            pltpu.VMEM((8, L), jnp.float32),
            pltpu.VMEM((8,), jnp.int32),
            pltpu.VMEM((8, L), jnp.float32),
            pltpu.VMEM((2 * N_NEIGHB + L,), jnp.float32),
            pltpu.VMEM((N_NEIGHB,), jnp.float32),
            [pltpu.VMEM((CW,), jnp.int32)] * N_CHUNKS,
            [pltpu.VMEM((CW,), jnp.float32)] * N_CHUNKS,
            pltpu.VMEM_SHARED((8, L), jnp.float32),
            [pltpu.SemaphoreType.DMA] * N_CHUNKS,
            [pltpu.SemaphoreType.DMA] * N_CHUNKS,
            pltpu.SemaphoreType.DMA,
        ),
    )
    return f(ind, ids, ch, pc)


def kernel(indicators, neighborhood_ids, channels, popcounts):
    cov, cvd, nsp, spike_cov = _run(
        indicators.astype(jnp.float32), neighborhood_ids.astype(jnp.int32),
        channels.astype(jnp.int32), popcounts.astype(jnp.int32))
    covered = cvd != 0.0
    n_spikes_covered = nsp[0].astype(jnp.int32)
    return cov, covered, n_spikes_covered, spike_cov
